# baseline (speedup 1.0000x reference)
import functools

import jax
import jax.numpy as jnp
from jax import lax
from jax.experimental import pallas as pl
from jax.experimental.pallas import tpu as pltpu
from jax.experimental.pallas import tpu_sc as plsc

VOCAB = 1000000
REL = 1000
DIM = 32
B = 16384

NC, NS = 2, 16
NW = NC * NS
RPW = B // NW
CHUNK = 128
NCHUNK = RPW // CHUNK


def _tanh(x):
    e = jnp.exp(2.0 * x)
    return 1.0 - 2.0 / (e + 1.0)


def _sqrt(x):
    i = plsc.bitcast(x, jnp.int32)
    y = plsc.bitcast(jnp.int32(0x5F3759DF) - (i >> 1), jnp.float32)
    for _ in range(3):
        y = y * (1.5 - 0.5 * x * y * y)
    return x * y


def _body(h_hbm, r_hbm, t_hbm, hp_hbm, rp_hbm, tp_hbm, hl_hbm, rl_hbm,
          o1_hbm, o2_hbm,
          hi_v, ri_v, ti_v, hpi_v, rpi_v, tpi_v,
          he_v, re_v, te_v, hpe_v, rpe_v, tpe_v,
          d1_v, d2_v, sem):
    wid = lax.axis_index("c") * NS + lax.axis_index("s")
    base = wid * RPW

    pltpu.sync_copy(h_hbm.at[pl.ds(base, RPW)], hi_v)
    pltpu.sync_copy(r_hbm.at[pl.ds(base, RPW)], ri_v)
    pltpu.sync_copy(t_hbm.at[pl.ds(base, RPW)], ti_v)
    pltpu.sync_copy(hp_hbm.at[pl.ds(base, RPW)], hpi_v)
    pltpu.sync_copy(rp_hbm.at[pl.ds(base, RPW)], rpi_v)
    pltpu.sync_copy(tp_hbm.at[pl.ds(base, RPW)], tpi_v)

    copies = []
    for c in range(NCHUNK):
        s = pl.ds(c * CHUNK, CHUNK)
        copies.append(pltpu.async_copy(hl_hbm.at[hi_v.at[s]], he_v.at[s], sem))
        copies.append(pltpu.async_copy(rl_hbm.at[ri_v.at[s]], re_v.at[s], sem))
        copies.append(pltpu.async_copy(hl_hbm.at[ti_v.at[s]], te_v.at[s], sem))
        copies.append(pltpu.async_copy(hl_hbm.at[hpi_v.at[s]], hpe_v.at[s], sem))
        copies.append(pltpu.async_copy(rl_hbm.at[rpi_v.at[s]], rpe_v.at[s], sem))
        copies.append(pltpu.async_copy(hl_hbm.at[tpi_v.at[s]], tpe_v.at[s], sem))
    for cp in copies:
        cp.wait()

    lanes = lax.iota(jnp.int32, 16)

    def triple(hrows, rrows, trows, dist_v):
        def group(g, _):
            rows = g * 16 + lanes

            def dim(j, acc):
                s_hh, s_rr, s_tt, s_ht, s_rth = acc
                cols = jnp.broadcast_to(j, (16,))
                hv = _tanh(plsc.load_gather(hrows, [rows, cols]))
                rv = _tanh(plsc.load_gather(rrows, [rows, cols]))
                tv = _tanh(plsc.load_gather(trows, [rows, cols]))
                return (s_hh + hv * hv, s_rr + rv * rv, s_tt + tv * tv,
                        s_ht + hv * tv, s_rth + rv * (tv - hv))

            z = jnp.zeros((16,), jnp.float32)
            s_hh, s_rr, s_tt, s_ht, s_rth = lax.fori_loop(
                0, DIM, dim, (z, z, z, z, z))
            dist = (_sqrt(s_hh) + _sqrt(s_rr) + _sqrt(s_tt)
                    - 2.0 * (s_ht + s_rth))
            plsc.store_scatter(dist_v, [rows], dist)
            return 0

        lax.fori_loop(0, RPW // 16, group, 0)

    triple(he_v, re_v, te_v, d1_v)
    triple(hpe_v, rpe_v, tpe_v, d2_v)

    pltpu.sync_copy(d1_v, o1_hbm.at[pl.ds(base, RPW)])
    pltpu.sync_copy(d2_v, o2_hbm.at[pl.ds(base, RPW)])


@functools.partial(jax.jit, static_argnames=())
def kernel(h, r, t, h_, r_, t_, hl, rl):
    mesh = plsc.VectorSubcoreMesh(core_axis_name="c", subcore_axis_name="s")
    f = pl.kernel(
        _body,
        out_type=(jax.ShapeDtypeStruct((B,), jnp.float32),
                  jax.ShapeDtypeStruct((B,), jnp.float32)),
        mesh=mesh,
        compiler_params=pltpu.CompilerParams(
            needs_layout_passes=False, use_tc_tiling_on_sc=False),
        scratch_types=(
            [pltpu.VMEM((RPW,), jnp.int32)] * 6
            + [pltpu.VMEM((RPW, DIM), jnp.float32)] * 6
            + [pltpu.VMEM((RPW,), jnp.float32)] * 2
            + [pltpu.SemaphoreType.DMA]
        ),
    )
    return f(h.astype(jnp.int32), r.astype(jnp.int32), t.astype(jnp.int32),
             h_.astype(jnp.int32), r_.astype(jnp.int32), t_.astype(jnp.int32),
             hl, rl)
